# Initial kernel scaffold; baseline (speedup 1.0000x reference)
import jax
import jax.numpy as jnp
from jax.experimental import pallas as pl


def kernel(seq, table):
    raise NotImplementedError("write your pallas kernel here")


# speedup vs baseline: 4.5819x; 4.5819x over previous
import functools

import jax
import jax.numpy as jnp
from jax import lax
from jax.experimental import pallas as pl
from jax.experimental.pallas import tpu as pltpu
from jax.experimental.pallas import tpu_sc as plsc

VOCAB = 32
D = 64
B = 16384
H = 200
N = B * H
IDXW = 128
NW = 32
ROWS_PER_W = N // (IDXW * NW)
GS = 4
NG = ROWS_PER_W // GS
NPAIR = GS * IDXW // 2


def _sc_gather(seq2d, table):
    mesh = plsc.VectorSubcoreMesh(core_axis_name="c", subcore_axis_name="s")

    @functools.partial(
        pl.kernel,
        out_type=jax.ShapeDtypeStruct((N // 2, 2 * D), jnp.float32),
        mesh=mesh,
        compiler_params=pltpu.CompilerParams(needs_layout_passes=False),
        scratch_types=[
            pltpu.HBM((2, VOCAB * VOCAB, 2 * D), jnp.float32),
            pltpu.VMEM((VOCAB, D), jnp.float32),
            pltpu.VMEM((2 * VOCAB, 2 * D), jnp.float32),
            pltpu.VMEM((GS, IDXW), jnp.int32),
            pltpu.VMEM((2, IDXW), jnp.int32),
            pltpu.VMEM((NPAIR, 2 * D), jnp.float32),
            pltpu.SemaphoreType.DMA,
        ],
    )
    def k(seq_hbm, table_hbm, out_hbm, pairs_hbm,
          table_v, build_v, sidx_v, pidx_v, got_v, sem):
        cid = lax.axis_index("c")
        sid = lax.axis_index("s")

        pltpu.sync_copy(table_hbm, table_v)
        for a in range(2):
            i0 = sid * 2 + a
            left = [table_v[i0, pl.ds(c * 16, 16)] for c in range(4)]
            for i1 in range(VOCAB):
                r = a * VOCAB + i1
                for c in range(4):
                    build_v[r, pl.ds(c * 16, 16)] = left[c]
                    build_v[r, pl.ds(D + c * 16, 16)] = table_v[i1, pl.ds(c * 16, 16)]
        pltpu.sync_copy(build_v, pairs_hbm.at[cid].at[pl.ds(sid * 2 * VOCAB, 2 * VOCAB)])
        plsc.subcore_barrier()

        wid = sid * 2 + cid
        base_row = wid * ROWS_PER_W
        lanes = lax.iota(jnp.int32, 16)

        @pl.loop(0, NG)
        def _group(g):
            r0 = base_row + g * GS
            pltpu.sync_copy(seq_hbm.at[pl.ds(r0, GS)], sidx_v)
            for rr in range(2):
                for b in range(2):
                    sr = 2 * rr + b
                    row_sel = jnp.full((16,), sr, dtype=jnp.int32)
                    for cc in range(4):
                        cols = lanes * 2 + (cc * 32)
                        ev = plsc.load_gather(sidx_v, [row_sel, cols])
                        od = plsc.load_gather(sidx_v, [row_sel, cols + 1])
                        pidx_v[rr, pl.ds(b * 64 + cc * 16, 16)] = ev * VOCAB + od
            for rr in range(2):
                pltpu.async_copy(
                    pairs_hbm.at[cid].at[pidx_v.at[rr]],
                    got_v.at[pl.ds(rr * IDXW, IDXW)],
                    sem,
                )
            for rr in range(2):
                pltpu.make_async_copy(
                    pairs_hbm.at[cid].at[pidx_v.at[rr]],
                    got_v.at[pl.ds(rr * IDXW, IDXW)],
                    sem,
                ).wait()
            pltpu.sync_copy(got_v, out_hbm.at[pl.ds(r0 * (IDXW // 2), NPAIR)])

    return k(seq2d, table)


def kernel(seq, table):
    seq2d = seq.reshape(N // IDXW, IDXW)
    out = _sc_gather(seq2d, table)
    return out.reshape(B, H, D)


# speedup vs baseline: 4.7972x; 1.0470x over previous
import functools

import jax
import jax.numpy as jnp
from jax import lax
from jax.experimental import pallas as pl
from jax.experimental.pallas import tpu as pltpu
from jax.experimental.pallas import tpu_sc as plsc

VOCAB = 32
D = 64
B = 16384
H = 200
N = B * H
IDXW = 128
NW = 32
ROWS_PER_W = N // (IDXW * NW)
GS = 4
NG = ROWS_PER_W // GS
NPAIR = GS * IDXW // 2
CH = 8
NCH = NG // CH
CROWS = CH * GS
PROWS = CROWS // 2


def _sc_gather(seq2d, table):
    mesh = plsc.VectorSubcoreMesh(core_axis_name="c", subcore_axis_name="s")

    @functools.partial(
        pl.kernel,
        out_type=jax.ShapeDtypeStruct((N // 2, 2 * D), jnp.float32),
        mesh=mesh,
        compiler_params=pltpu.CompilerParams(needs_layout_passes=False),
        scratch_types=[
            pltpu.HBM((2, VOCAB * VOCAB, 2 * D), jnp.float32),
            pltpu.VMEM((VOCAB, D), jnp.float32),
            pltpu.VMEM((2 * VOCAB, 2 * D), jnp.float32),
            pltpu.VMEM((2, CROWS, IDXW), jnp.int32),
            pltpu.VMEM((2, PROWS, IDXW), jnp.int32),
            pltpu.VMEM((2, NPAIR, 2 * D), jnp.float32),
            pltpu.SemaphoreType.DMA,
            pltpu.SemaphoreType.DMA,
            pltpu.SemaphoreType.DMA,
        ],
    )
    def k(seq_hbm, table_hbm, out_hbm, pairs_hbm,
          table_v, build_v, sidx_v, pidx_v, got_v, gsem, wsem, isem):
        cid = lax.axis_index("c")
        sid = lax.axis_index("s")

        pltpu.sync_copy(table_hbm, table_v)
        for a in range(2):
            i0 = sid * 2 + a
            left = [table_v[i0, pl.ds(c * 16, 16)] for c in range(4)]
            for i1 in range(VOCAB):
                r = a * VOCAB + i1
                for c in range(4):
                    build_v[r, pl.ds(c * 16, 16)] = left[c]
                    build_v[r, pl.ds(D + c * 16, 16)] = table_v[i1, pl.ds(c * 16, 16)]
        pltpu.sync_copy(build_v, pairs_hbm.at[cid].at[pl.ds(sid * 2 * VOCAB, 2 * VOCAB)])
        plsc.subcore_barrier()

        wid = sid * 2 + cid
        base_row = wid * ROWS_PER_W
        lanes = lax.iota(jnp.int32, 16)
        zeros16 = jnp.zeros((16,), jnp.int32)

        def idx_copy(ch, buf):
            return pltpu.make_async_copy(
                seq_hbm.at[pl.ds(base_row + ch * CROWS, CROWS)],
                sidx_v.at[buf], isem)

        def gather_copy(grp, rr, b):
            ch = grp // CH
            prow = (grp % CH) * 2 + rr
            return pltpu.make_async_copy(
                pairs_hbm.at[cid].at[pidx_v.at[ch % 2].at[prow]],
                got_v.at[b].at[pl.ds(rr * IDXW, IDXW)], gsem)

        def write_copy(grp, b):
            return pltpu.make_async_copy(
                got_v.at[b], out_hbm.at[pl.ds(grp * NPAIR, NPAIR)], wsem)

        def fire_gathers(grp, b):
            for rr in range(2):
                gather_copy(grp, rr, b).start()

        def drain_gathers(grp, b):
            for rr in range(2):
                gather_copy(grp, rr, b).wait()

        idx_copy(0, 0).start()

        @pl.loop(0, NCH)
        def _chunk(ch):
            sb = lax.rem(ch, 2)
            idx_copy(ch, sb).wait()

            @pl.when(ch + 1 < NCH)
            def _():
                idx_copy(ch + 1, 1 - sb).start()

            sb_vec = zeros16 + sb
            for prow in range(PROWS):
                for hh in range(2):
                    sr_vec = zeros16 + (2 * prow + hh)
                    for cc in range(4):
                        cols = lanes * 2 + cc * 32
                        ev = plsc.load_gather(sidx_v, [sb_vec, sr_vec, cols])
                        od = plsc.load_gather(sidx_v, [sb_vec, sr_vec, cols + 1])
                        pidx_v[sb, prow, pl.ds(hh * 64 + cc * 16, 16)] = ev * VOCAB + od

            for gg in range(CH):
                b = gg % 2
                grp = ch * CH + gg

                @pl.when(grp >= 2)
                def _():
                    write_copy(0, b).wait()

                fire_gathers(grp, b)

                if gg == 0:
                    @pl.when(ch > 0)
                    def _():
                        drain_gathers(ch * CH - 1, 1)
                        write_copy(ch * CH - 1, 1).start()
                else:
                    drain_gathers(grp - 1, 1 - b)
                    write_copy(grp - 1, 1 - b).start()

        drain_gathers(NG - 1, 1)
        write_copy(NG - 1, 1).start()
        write_copy(0, 0).wait()
        write_copy(0, 1).wait()

    return k(seq2d, table)


def kernel(seq, table):
    seq2d = seq.reshape(N // IDXW, IDXW)
    out = _sc_gather(seq2d, table)
    return out.reshape(B, H, D)


# speedup vs baseline: 9.1673x; 1.9110x over previous
import functools

import jax
import jax.numpy as jnp
from jax import lax
from jax.experimental import pallas as pl
from jax.experimental.pallas import tpu as pltpu
from jax.experimental.pallas import tpu_sc as plsc

VOCAB = 32
D = 64
B = 16384
H = 200
NW = 32
BW = B // NW
CH_H = 8
NCH = H // CH_H
NC16 = BW // 16


def _sc_embed(seq_t, table):
    mesh = plsc.VectorSubcoreMesh(core_axis_name="c", subcore_axis_name="s")

    @functools.partial(
        pl.kernel,
        out_type=jax.ShapeDtypeStruct((H * D, B), jnp.float32),
        mesh=mesh,
        compiler_params=pltpu.CompilerParams(needs_layout_passes=False),
        scratch_types=[
            pltpu.VMEM((VOCAB, D), jnp.float32),
            pltpu.VMEM((VOCAB * D,), jnp.float32),
            pltpu.VMEM((2, CH_H, BW), jnp.int32),
            pltpu.VMEM((2, D, BW), jnp.float32),
            pltpu.SemaphoreType.DMA,
            pltpu.SemaphoreType.DMA,
        ],
    )
    def k(seq_hbm, table_hbm, out_hbm, table_v, lut_v, sidx_v, obuf_v,
          wsem, isem):
        cid = lax.axis_index("c")
        sid = lax.axis_index("s")
        wid = sid * 2 + cid
        b0 = wid * BW
        lanes = lax.iota(jnp.int32, 16)

        pltpu.sync_copy(table_hbm, table_v)
        for v in range(VOCAB):
            for c in range(4):
                x = table_v[v, pl.ds(c * 16, 16)]
                dix = (lanes + c * 16) * VOCAB + v
                plsc.store_scatter(lut_v, [dix], x)

        def idx_copy(ch, buf):
            return pltpu.make_async_copy(
                seq_hbm.at[pl.ds(ch * CH_H, CH_H), pl.ds(b0, BW)],
                sidx_v.at[buf], isem)

        def write_copy(h, buf):
            return pltpu.make_async_copy(
                obuf_v.at[buf], out_hbm.at[pl.ds(h * D, D), pl.ds(b0, BW)],
                wsem)

        idx_copy(0, 0).start()

        @pl.loop(0, NCH)
        def _chunk(ch):
            sb = lax.rem(ch, 2)
            idx_copy(ch, sb).wait()

            @pl.when(ch + 1 < NCH)
            def _():
                idx_copy(ch + 1, 1 - sb).start()

            for hl in range(CH_H):
                h = ch * CH_H + hl
                buf = hl % 2
                if hl >= 2:
                    write_copy(0, buf).wait()
                else:
                    @pl.when(ch > 0)
                    def _():
                        write_copy(0, buf).wait()

                idxs = [sidx_v[sb, hl, pl.ds(c * 16, 16)] for c in range(NC16)]

                @pl.loop(0, D, unroll=4)
                def _row(d):
                    base = d * VOCAB
                    for c in range(NC16):
                        g = plsc.load_gather(lut_v, [idxs[c] + base])
                        obuf_v[buf, d, pl.ds(c * 16, 16)] = g

                write_copy(h, buf).start()

        for _ in range(2):
            write_copy(0, 0).wait()

    return k(seq_t, table)


def kernel(seq, table):
    out2d = _sc_embed(jnp.swapaxes(seq, 0, 1), table)
    return out2d.reshape(H, D, B).transpose(2, 0, 1)


# speedup vs baseline: 37.6967x; 4.1121x over previous
import functools

import jax
import jax.numpy as jnp
from jax import lax
from jax.experimental import pallas as pl
from jax.experimental.pallas import tpu as pltpu
from jax.experimental.pallas import tpu_sc as plsc

VOCAB = 32
D = 64
B = 16384
H = 200
NW = 32
BW = B // NW
CH_H = 8
NCH = H // CH_H
NC16 = BW // 16


def _sc_embed(seq_t, table):
    mesh = plsc.VectorSubcoreMesh(core_axis_name="c", subcore_axis_name="s")

    @functools.partial(
        pl.kernel,
        out_type=jax.ShapeDtypeStruct((H * D, B), jnp.float32),
        mesh=mesh,
        compiler_params=pltpu.CompilerParams(needs_layout_passes=False),
        scratch_types=[
            pltpu.VMEM((VOCAB, D), jnp.float32),
            pltpu.VMEM((VOCAB * D,), jnp.float32),
            pltpu.VMEM((2, CH_H, BW), jnp.int32),
            pltpu.VMEM((2, D, BW), jnp.float32),
            pltpu.SemaphoreType.DMA,
            pltpu.SemaphoreType.DMA,
        ],
    )
    def k(seq_hbm, table_hbm, out_hbm, table_v, lut_v, sidx_v, obuf_v,
          wsem, isem):
        cid = lax.axis_index("c")
        sid = lax.axis_index("s")
        wid = sid * 2 + cid
        b0 = wid * BW
        lanes = lax.iota(jnp.int32, 16)

        pltpu.sync_copy(table_hbm, table_v)
        for v in range(VOCAB):
            for c in range(4):
                x = table_v[v, pl.ds(c * 16, 16)]
                dix = (lanes + c * 16) * VOCAB + v
                plsc.store_scatter(lut_v, [dix], x)

        def idx_copy(ch, buf):
            return pltpu.make_async_copy(
                seq_hbm.at[pl.ds(ch * CH_H, CH_H), pl.ds(b0, BW)],
                sidx_v.at[buf], isem)

        def write_copy(h, buf):
            return pltpu.make_async_copy(
                obuf_v.at[buf], out_hbm.at[pl.ds(h * D, D), pl.ds(b0, BW)],
                wsem)

        idx_copy(0, 0).start()

        @pl.loop(0, NCH)
        def _chunk(ch):
            sb = lax.rem(ch, 2)
            idx_copy(ch, sb).wait()

            @pl.when(ch + 1 < NCH)
            def _():
                idx_copy(ch + 1, 1 - sb).start()

            for hl in range(CH_H):
                h = ch * CH_H + hl
                buf = hl % 2
                if hl >= 2:
                    write_copy(0, buf).wait()
                else:
                    @pl.when(ch > 0)
                    def _():
                        write_copy(0, buf).wait()

                idxs = [sidx_v[sb, hl, pl.ds(c * 16, 16)] for c in range(NC16)]

                @plsc.parallel_loop(0, D, unroll=4)
                def _row(d):
                    lut_d = lut_v.at[pl.ds(d * VOCAB, VOCAB)]
                    for c in range(NC16):
                        g = plsc.load_gather(lut_d, [idxs[c]])
                        obuf_v[buf, d, pl.ds(c * 16, 16)] = g

                write_copy(h, buf).start()

        for _ in range(2):
            write_copy(0, 0).wait()

    return k(seq_t, table)


def kernel(seq, table):
    out2d = _sc_embed(jnp.swapaxes(seq, 0, 1), table)
    return out2d.reshape(H, D, B).transpose(2, 0, 1)
